# baseline (speedup 1.0000x reference)
import functools

import jax
import jax.numpy as jnp
from jax import lax
from jax.experimental import pallas as pl
from jax.experimental.pallas import tpu as pltpu
from jax.experimental.pallas import tpu_sc as plsc

H = 128
N_TASK = 10000
N_RET = 4096
N_DRAM = 1024
N_LINK = 8192
E = 320000

NC = 2
NS = 16
L = 16
MT = 10240
ACC_T = N_TASK * 8
BLK_T = MT * 8
NLH = N_LINK // 2
ACC_L = NLH * 8

BQ = 8
_NEG = -3.0e38
_TCUT = -1.0e37

_MESH = plsc.VectorSubcoreMesh(
    core_axis_name="c", subcore_axis_name="s", num_cores=NC, num_subcores=NS)
_SC_PARAMS = pltpu.CompilerParams(needs_layout_passes=False)


def _iota():
    return lax.iota(jnp.int32, L)


def _vperm(x, idx):
    dnums = lax.GatherDimensionNumbers(
        offset_dims=(), collapsed_slice_dims=(0,), start_index_map=(0,))
    return lax.gather(x, idx[:, None], dnums, (1,),
                      mode=lax.GatherScatterMode.PROMISE_IN_BOUNDS)


def _sc_segsum(f_ret, d_ret, f_dram, d_dram, f_lnk, d_lnk, z1):
    epw = E // (NC * NS)
    W = 2000
    nw = epw // W

    @functools.partial(
        pl.kernel,
        out_type=[
            jax.ShapeDtypeStruct((NC * 2 * N_RET,), jnp.float32),
            jax.ShapeDtypeStruct((NC * 2 * N_DRAM,), jnp.float32),
            jax.ShapeDtypeStruct((NC * 2 * N_LINK,), jnp.float32),
        ],
        mesh=_MESH,
        scratch_types=[
            pltpu.VMEM((2 * W,), jnp.float32),
            pltpu.VMEM((W,), jnp.int32),
            pltpu.VMEM((2 * W,), jnp.int32),
            pltpu.VMEM_SHARED((2 * N_RET,), jnp.float32),
            pltpu.VMEM_SHARED((2 * N_DRAM,), jnp.float32),
            pltpu.VMEM_SHARED((2 * N_LINK,), jnp.float32),
        ],
        compiler_params=_SC_PARAMS,
    )
    def k(fr, dr, fd, dd, fl, dl, zz, p_ret, p_dram, p_lnk,
          fwin, iwin, idx2, s_ret, s_dram, s_lnk):
        cid = lax.axis_index("c")
        sid = lax.axis_index("s")
        w = cid * NS + sid
        iot = _iota()

        @pl.when(sid == 0)
        def _zero():
            for s_acc, n2 in ((s_ret, 2 * N_RET), (s_dram, 2 * N_DRAM),
                              (s_lnk, 2 * N_LINK)):
                for r in range(0, n2, 2048):
                    pltpu.sync_copy(zz, s_acc.at[pl.ds(r, 2048)])

        plsc.subcore_barrier()

        for feat, dix, s_acc in ((fr, dr, s_ret), (fd, dd, s_dram),
                                 (fl, dl, s_lnk)):
            def wbody(kw, _, feat=feat, dix=dix, s_acc=s_acc):
                b = w * epw + kw * W
                pltpu.sync_copy(feat.at[pl.ds(2 * b, 2 * W)], fwin)
                pltpu.sync_copy(dix.at[pl.ds(b, W)], iwin)

                def ibody(i, _):
                    j = i * L + iot
                    d = plsc.load_gather(iwin, [j >> 1])
                    plsc.store_scatter(idx2, [j], d * 2 + (j & 1))
                    return 0
                lax.fori_loop(0, (2 * W) // L, ibody, 0, unroll=4)
                pltpu.sync_copy(fwin, s_acc.at[idx2], add=True)
                return 0
            lax.fori_loop(0, nw, wbody, 0)

        plsc.subcore_barrier()

        @pl.when(sid == 0)
        def _out():
            pltpu.sync_copy(s_ret, p_ret.at[pl.ds(cid * 2 * N_RET, 2 * N_RET)])
            pltpu.sync_copy(s_dram,
                            p_dram.at[pl.ds(cid * 2 * N_DRAM, 2 * N_DRAM)])
            pltpu.sync_copy(s_lnk, p_lnk.at[pl.ds(cid * 2 * N_LINK, 2 * N_LINK)])

    return k(f_ret, d_ret, f_dram, d_dram, f_lnk, d_lnk, z1)


def _sc_segmax3(p_ret, p_dram, p_lnk, wp1, src_ret, dst_ret, src_dram,
                dst_dram, src_lnk, dst_lnk, neg1):
    EH = E // 2
    W = 4000
    nw = EH // W

    @functools.partial(
        pl.kernel,
        out_type=[
            jax.ShapeDtypeStruct((NC * NS * BLK_T,), jnp.float32),
            jax.ShapeDtypeStruct((NC * NS * BLK_T,), jnp.float32),
            jax.ShapeDtypeStruct((NC * NS * BLK_T,), jnp.float32),
        ],
        mesh=_MESH,
        scratch_types=[
            pltpu.VMEM((ACC_T,), jnp.float32),
            pltpu.VMEM((N_LINK,), jnp.float32),
            pltpu.VMEM((N_LINK,), jnp.float32),
            pltpu.VMEM((2 * N_LINK,), jnp.float32),
            pltpu.VMEM((W,), jnp.int32),
            pltpu.VMEM((W,), jnp.int32),
            pltpu.VMEM((3 * 2 * NS * L,), jnp.float32),
        ],
        compiler_params=_SC_PARAMS,
    )
    def k(pr, pd, plk, wpr, sr, dr, sd, dd, sl, dl, ng,
          m_ret, m_dram, m_lnk,
          acc, u_tab, v_tab, pbuf, win_s, win_d, wbuf):
        cid = lax.axis_index("c")
        sid = lax.axis_index("s")
        iot = _iota()
        coloff = iot & 7
        c0 = iot >> 3
        fh = iot < 8
        swp = (iot + 8) & 15

        pltpu.sync_copy(wpr, wbuf)

        cases = (
            (pr, sr, dr, N_RET, 0, m_ret),
            (pd, sd, dd, N_DRAM, 1, m_dram),
            (plk, sl, dl, N_LINK, 2, m_lnk),
        )
        for p_hbm, s_hbm, d_hbm, n, t, m_out in cases:
            for hp in range(NC):
                pltpu.sync_copy(p_hbm.at[pl.ds(hp * 2 * n, 2 * n)],
                                pbuf.at[pl.ds(0, 2 * n)])

                def dbody(i, _, first=(hp == 0)):
                    r = i * L + iot
                    uu = plsc.load_gather(pbuf, [r * 2])
                    vv = plsc.load_gather(pbuf, [r * 2 + 1])
                    if not first:
                        uu = uu + plsc.load_gather(u_tab, [r])
                        vv = vv + plsc.load_gather(v_tab, [r])
                    plsc.store_scatter(u_tab, [r], uu)
                    plsc.store_scatter(v_tab, [r], vv)
                    return 0
                lax.fori_loop(0, n // L, dbody, 0, unroll=4)

            w0 = plsc.load_gather(wbuf, [(t * 2 + 0) * NS * L + sid * L + iot])
            w1 = plsc.load_gather(wbuf, [(t * 2 + 1) * NS * L + sid * L + iot])

            pltpu.sync_copy(ng.at[pl.ds(0, ACC_T)], acc)

            def ebody(j, cidx):
                ids = plsc.load_gather(win_s, [j * (2 * BQ) + iot])
                dup = jnp.zeros((L,), jnp.bool_)
                for r in range(1, 9):
                    dup = jnp.logical_or(dup,
                                         ids == _vperm(ids, (iot + r) & 15))
                nodup = jnp.logical_not(jnp.any(dup))
                work = []
                for bq in range(BQ):
                    ci = cidx + 2 * bq
                    spair = plsc.load_gather(win_s, [ci])
                    dpair = plsc.load_gather(win_d, [ci])
                    u = plsc.load_gather(u_tab, [dpair])
                    v = plsc.load_gather(v_tab, [dpair])
                    z = u * w0 + v * w1
                    sswap = _vperm(spair, swp)
                    eq = spair == sswap
                    zswap = _vperm(z, swp)
                    z2 = jnp.where(eq, jnp.maximum(z, zswap), z)
                    mask = jnp.logical_or(jnp.logical_not(eq), fh)
                    work.append((spair * 8 + coloff, z2, mask))

                @pl.when(nodup)
                def _fast():
                    curs = [plsc.load_gather(acc, [a], mask=m)
                            for a, _, m in work]
                    for (addr, z2, mask), cur in zip(work, curs):
                        plsc.store_scatter(acc, [addr],
                                           jnp.maximum(cur, z2), mask=mask)

                @pl.when(jnp.logical_not(nodup))
                def _slow():
                    for addr, z2, mask in work:
                        cur = plsc.load_gather(acc, [addr], mask=mask)
                        plsc.store_scatter(acc, [addr],
                                           jnp.maximum(cur, z2), mask=mask)
                return cidx + 2 * BQ

            def wbody(kw, _, s_hbm=s_hbm, d_hbm=d_hbm):
                b = cid * EH + kw * W
                pltpu.sync_copy(s_hbm.at[pl.ds(b, W)], win_s)
                pltpu.sync_copy(d_hbm.at[pl.ds(b, W)], win_d)
                lax.fori_loop(0, W // (2 * BQ), ebody, c0)
                return 0
            lax.fori_loop(0, nw, wbody, 0)

            blk = (cid * NS + sid) * BLK_T
            pltpu.sync_copy(acc, m_out.at[pl.ds(blk, ACC_T)])

    return k(p_ret, p_dram, p_lnk, wp1, src_ret, dst_ret, src_dram, dst_dram,
             src_lnk, dst_lnk, neg1)


def _sc_segmax_link(th1, src_lnk, dst_lnk, neg1):
    W = 4000
    nw = E // W

    @functools.partial(
        pl.kernel,
        out_type=jax.ShapeDtypeStruct((NC * NS * ACC_L,), jnp.float32),
        mesh=_MESH,
        scratch_types=[
            pltpu.VMEM((ACC_T,), jnp.float32),
            pltpu.VMEM((ACC_L,), jnp.float32),
            pltpu.VMEM((W,), jnp.int32),
            pltpu.VMEM((W,), jnp.int32),
        ],
        compiler_params=_SC_PARAMS,
    )
    def k(th, sl, dl, ng, mm, table, acc, win_s, win_d):
        cid = lax.axis_index("c")
        sid = lax.axis_index("s")
        iot = _iota()
        coloff = iot & 7
        c0 = iot >> 3
        fh = iot < 8
        swp = (iot + 8) & 15
        lo = jnp.full((L,), cid * NLH, jnp.int32)

        pltpu.sync_copy(th.at[pl.ds(sid * BLK_T, ACC_T)], table)
        pltpu.sync_copy(ng.at[pl.ds(0, ACC_L)], acc)

        def ebody(j, cidx):
            ids = plsc.load_gather(win_d, [j * (2 * BQ) + iot])
            dup = jnp.zeros((L,), jnp.bool_)
            for r in range(1, 9):
                dup = jnp.logical_or(dup,
                                     ids == _vperm(ids, (iot + r) & 15))
            nodup = jnp.logical_not(jnp.any(dup))
            work = []
            for bq in range(BQ):
                ci = cidx + 2 * bq
                tpair = plsc.load_gather(win_s, [ci])
                lpair = plsc.load_gather(win_d, [ci])
                z = plsc.load_gather(table, [tpair * 8 + coloff])
                lswap = _vperm(lpair, swp)
                eq = lpair == lswap
                zswap = _vperm(z, swp)
                z2 = jnp.where(eq, jnp.maximum(z, zswap), z)
                row = lpair - lo
                inr = jnp.logical_and(row >= 0, row < NLH)
                rowc = jnp.clip(row, 0, NLH - 1)
                mask = jnp.logical_and(
                    inr, jnp.logical_or(jnp.logical_not(eq), fh))
                work.append((rowc * 8 + coloff, z2, mask))

            @pl.when(nodup)
            def _fast():
                curs = [plsc.load_gather(acc, [a], mask=m)
                        for a, _, m in work]
                for (addr, z2, mask), cur in zip(work, curs):
                    plsc.store_scatter(acc, [addr],
                                       jnp.maximum(cur, z2), mask=mask)

            @pl.when(jnp.logical_not(nodup))
            def _slow():
                for addr, z2, mask in work:
                    cur = plsc.load_gather(acc, [addr], mask=mask)
                    plsc.store_scatter(acc, [addr],
                                       jnp.maximum(cur, z2), mask=mask)
            return cidx + 2 * BQ

        def wbody(kw, _):
            b = kw * W
            pltpu.sync_copy(sl.at[pl.ds(b, W)], win_s)
            pltpu.sync_copy(dl.at[pl.ds(b, W)], win_d)
            lax.fori_loop(0, W // (2 * BQ), ebody, c0)
            return 0
        lax.fori_loop(0, nw, wbody, 0)

        blk = (cid * NS + sid) * ACC_L
        pltpu.sync_copy(acc, mm.at[pl.ds(blk, ACC_L)])

    return k(th1, src_lnk, dst_lnk, neg1)


def _gelu_ln(y, g, beta):
    y = y * 0.5 * (1.0 + lax.erf(y * 0.7071067811865476))
    mu = jnp.mean(y, axis=-1, keepdims=True)
    var = jnp.mean((y - mu) ** 2, axis=-1, keepdims=True)
    return (y - mu) / jnp.sqrt(var + 1e-5) * g + beta


def _task_body(mr_ref, md_ref, ml_ref, br_ref, bd_ref, bl_ref,
               wt_ref, bt_ref, g_ref, beta_ref, o_ref):
    hs = []
    for m_ref, b_ref in ((mr_ref, br_ref), (md_ref, bd_ref), (ml_ref, bl_ref)):
        mm = jnp.maximum(m_ref[0], m_ref[1])
        hs.append(jnp.where(mm > _TCUT, jnp.tanh(mm + b_ref[...]), 0.0))
    t = jnp.concatenate(hs, axis=-1)
    y = t @ wt_ref[...] + bt_ref[...]
    o_ref[...] = _gelu_ln(y, g_ref[...], beta_ref[...])


def _tc_task(m_ret, m_dram, m_lnk, b_ret, b_dram, b_lnk,
             W_task, b_task, g_task, beta_task):
    R = 512
    vec = pl.BlockSpec((1, H), lambda i: (0, 0))
    return pl.pallas_call(
        _task_body,
        grid=(MT // R,),
        in_specs=[
            pl.BlockSpec((NC, R, H), lambda i: (0, i, 0)),
            pl.BlockSpec((NC, R, H), lambda i: (0, i, 0)),
            pl.BlockSpec((NC, R, H), lambda i: (0, i, 0)),
            vec, vec, vec,
            pl.BlockSpec((3 * H, H), lambda i: (0, 0)),
            vec, vec, vec,
        ],
        out_specs=pl.BlockSpec((R, H), lambda i: (i, 0)),
        out_shape=jax.ShapeDtypeStruct((MT, H), jnp.float32),
    )(m_ret, m_dram, m_lnk, b_ret, b_dram, b_lnk,
      W_task, b_task, g_task, beta_task)


def _link_body(m_ref, w_ref, b_ref, g_ref, beta_ref, o_ref):
    x = jnp.where(m_ref[...] > _TCUT, m_ref[...], 0.0)
    y = x @ w_ref[...] + b_ref[...]
    o_ref[...] = _gelu_ln(y, g_ref[...], beta_ref[...])


def _tc_link(m_mod, W_mod, b_mod, g_link, beta_link):
    R = 512
    vec = pl.BlockSpec((1, H), lambda i: (0, 0))
    return pl.pallas_call(
        _link_body,
        grid=(N_LINK // R,),
        in_specs=[
            pl.BlockSpec((R, H), lambda i: (i, 0)),
            pl.BlockSpec((H, H), lambda i: (0, 0)),
            vec, vec, vec,
        ],
        out_specs=pl.BlockSpec((R, H), lambda i: (i, 0)),
        out_shape=jax.ShapeDtypeStruct((N_LINK, H), jnp.float32),
    )(m_mod, W_mod, b_mod, g_link, beta_link)


def _wpair(w):
    return jnp.tile(w.reshape(2, NS, 8), (1, 1, 2))


def _asm_task(m1):
    m4 = m1.reshape(NC, NS, MT, 8)
    return m4.transpose(0, 2, 1, 3).reshape(NC, MT, H)


def kernel(feat_reticle, feat_dram_port, feat_link, src_reticle, dst_reticle,
           src_dram_port, dst_dram_port, src_link, dst_link,
           W_ret, b_ret, W_dram, b_dram, W_lnk, b_lnk,
           W_task, b_task, W_mod, b_mod,
           g_task, beta_task, g_link, beta_link):
    z1 = jnp.zeros((2048,), jnp.float32)
    neg1 = jnp.full((ACC_T,), _NEG, jnp.float32)
    wp1 = jnp.stack(
        [_wpair(W_ret), _wpair(W_dram), _wpair(W_lnk)]).reshape(-1)

    p_ret, p_dram, p_lnk = _sc_segsum(
        feat_reticle.reshape(-1), dst_reticle,
        feat_dram_port.reshape(-1), dst_dram_port,
        feat_link.reshape(-1), dst_link, z1)

    m1_ret, m1_dram, m1_lnk = _sc_segmax3(
        p_ret, p_dram, p_lnk, wp1, src_reticle, dst_reticle,
        src_dram_port, dst_dram_port, src_link, dst_link, neg1)

    task_h = _tc_task(
        _asm_task(m1_ret), _asm_task(m1_dram), _asm_task(m1_lnk),
        b_ret.reshape(1, H), b_dram.reshape(1, H), b_lnk.reshape(1, H),
        W_task, b_task.reshape(1, H), g_task.reshape(1, H),
        beta_task.reshape(1, H))

    th1 = task_h.reshape(MT, NS, 8).transpose(1, 0, 2).reshape(-1)

    mm1 = _sc_segmax_link(th1, src_link, dst_link, neg1)
    m_mod = mm1.reshape(NC, NS, NLH, 8).transpose(0, 2, 1, 3).reshape(
        N_LINK, H)

    link_h = _tc_link(
        m_mod, W_mod, b_mod.reshape(1, H), g_link.reshape(1, H),
        beta_link.reshape(1, H))
    return link_h


# speedup vs baseline: 1.2875x; 1.2875x over previous
import functools

import jax
import jax.numpy as jnp
from jax import lax
from jax.experimental import pallas as pl
from jax.experimental.pallas import tpu as pltpu
from jax.experimental.pallas import tpu_sc as plsc

H = 128
N_TASK = 10000
N_RET = 4096
N_DRAM = 1024
N_LINK = 8192
E = 320000

NC = 2
NS = 16
L = 16
MT = 10240
ACC_T = N_TASK * 8
BLK_T = MT * 8
NLH = N_LINK // 2
ACC_L = NLH * 8

BQ = 16
_NEG = -3.0e38
_TCUT = -1.0e37

_MESH = plsc.VectorSubcoreMesh(
    core_axis_name="c", subcore_axis_name="s", num_cores=NC, num_subcores=NS)
_SC_PARAMS = pltpu.CompilerParams(needs_layout_passes=False)


def _iota():
    return lax.iota(jnp.int32, L)


def _vperm(x, idx):
    dnums = lax.GatherDimensionNumbers(
        offset_dims=(), collapsed_slice_dims=(0,), start_index_map=(0,))
    return lax.gather(x, idx[:, None], dnums, (1,),
                      mode=lax.GatherScatterMode.PROMISE_IN_BOUNDS)


def _sc_segsum(f_ret, d_ret, f_dram, d_dram, f_lnk, d_lnk, z1):
    epw = E // (NC * NS)
    W = 2000
    nw = epw // W

    @functools.partial(
        pl.kernel,
        out_type=[
            jax.ShapeDtypeStruct((NC * 2 * N_RET,), jnp.float32),
            jax.ShapeDtypeStruct((NC * 2 * N_DRAM,), jnp.float32),
            jax.ShapeDtypeStruct((NC * 2 * N_LINK,), jnp.float32),
        ],
        mesh=_MESH,
        scratch_types=[
            pltpu.VMEM((2 * W,), jnp.float32),
            pltpu.VMEM((W,), jnp.int32),
            pltpu.VMEM((2 * W,), jnp.int32),
            pltpu.VMEM_SHARED((2 * N_RET,), jnp.float32),
            pltpu.VMEM_SHARED((2 * N_DRAM,), jnp.float32),
            pltpu.VMEM_SHARED((2 * N_LINK,), jnp.float32),
        ],
        compiler_params=_SC_PARAMS,
    )
    def k(fr, dr, fd, dd, fl, dl, zz, p_ret, p_dram, p_lnk,
          fwin, iwin, idx2, s_ret, s_dram, s_lnk):
        cid = lax.axis_index("c")
        sid = lax.axis_index("s")
        w = cid * NS + sid
        iot = _iota()

        @pl.when(sid == 0)
        def _zero():
            for s_acc, n2 in ((s_ret, 2 * N_RET), (s_dram, 2 * N_DRAM),
                              (s_lnk, 2 * N_LINK)):
                for r in range(0, n2, 2048):
                    pltpu.sync_copy(zz, s_acc.at[pl.ds(r, 2048)])

        plsc.subcore_barrier()

        for feat, dix, s_acc in ((fr, dr, s_ret), (fd, dd, s_dram),
                                 (fl, dl, s_lnk)):
            def wbody(kw, _, feat=feat, dix=dix, s_acc=s_acc):
                b = w * epw + kw * W
                pltpu.sync_copy(feat.at[pl.ds(2 * b, 2 * W)], fwin)
                pltpu.sync_copy(dix.at[pl.ds(b, W)], iwin)

                def ibody(i, _):
                    j = i * L + iot
                    d = plsc.load_gather(iwin, [j >> 1])
                    plsc.store_scatter(idx2, [j], d * 2 + (j & 1))
                    return 0
                lax.fori_loop(0, (2 * W) // L, ibody, 0, unroll=4)
                pltpu.sync_copy(fwin, s_acc.at[idx2], add=True)
                return 0
            lax.fori_loop(0, nw, wbody, 0)

        plsc.subcore_barrier()

        @pl.when(sid == 0)
        def _out():
            pltpu.sync_copy(s_ret, p_ret.at[pl.ds(cid * 2 * N_RET, 2 * N_RET)])
            pltpu.sync_copy(s_dram,
                            p_dram.at[pl.ds(cid * 2 * N_DRAM, 2 * N_DRAM)])
            pltpu.sync_copy(s_lnk, p_lnk.at[pl.ds(cid * 2 * N_LINK, 2 * N_LINK)])

    return k(f_ret, d_ret, f_dram, d_dram, f_lnk, d_lnk, z1)


def _sc_segmax3(p_ret, p_dram, p_lnk, wp1, src_ret, dst_ret, src_dram,
                dst_dram, src_lnk, dst_lnk, neg1):
    EH = E // 2
    W = 8000
    nw = EH // W

    @functools.partial(
        pl.kernel,
        out_type=[
            jax.ShapeDtypeStruct((NC * NS * BLK_T,), jnp.float32),
            jax.ShapeDtypeStruct((NC * NS * BLK_T,), jnp.float32),
            jax.ShapeDtypeStruct((NC * NS * BLK_T,), jnp.float32),
        ],
        mesh=_MESH,
        scratch_types=[
            pltpu.VMEM((ACC_T,), jnp.float32),
            pltpu.VMEM((N_LINK,), jnp.float32),
            pltpu.VMEM((N_LINK,), jnp.float32),
            pltpu.VMEM((2 * N_LINK,), jnp.float32),
            pltpu.VMEM((W,), jnp.int32),
            pltpu.VMEM((W,), jnp.int32),
            pltpu.VMEM((3 * 2 * NS * L,), jnp.float32),
        ],
        compiler_params=_SC_PARAMS,
    )
    def k(pr, pd, plk, wpr, sr, dr, sd, dd, sl, dl, ng,
          m_ret, m_dram, m_lnk,
          acc, u_tab, v_tab, pbuf, win_s, win_d, wbuf):
        cid = lax.axis_index("c")
        sid = lax.axis_index("s")
        iot = _iota()
        coloff = iot & 7
        c0 = iot >> 3
        fh = iot < 8
        swp = (iot + 8) & 15

        pltpu.sync_copy(wpr, wbuf)

        cases = (
            (pr, sr, dr, N_RET, 0, m_ret),
            (pd, sd, dd, N_DRAM, 1, m_dram),
            (plk, sl, dl, N_LINK, 2, m_lnk),
        )
        for p_hbm, s_hbm, d_hbm, n, t, m_out in cases:
            for hp in range(NC):
                pltpu.sync_copy(p_hbm.at[pl.ds(hp * 2 * n, 2 * n)],
                                pbuf.at[pl.ds(0, 2 * n)])

                def dbody(i, _, first=(hp == 0)):
                    r = i * L + iot
                    uu = plsc.load_gather(pbuf, [r * 2])
                    vv = plsc.load_gather(pbuf, [r * 2 + 1])
                    if not first:
                        uu = uu + plsc.load_gather(u_tab, [r])
                        vv = vv + plsc.load_gather(v_tab, [r])
                    plsc.store_scatter(u_tab, [r], uu)
                    plsc.store_scatter(v_tab, [r], vv)
                    return 0
                lax.fori_loop(0, n // L, dbody, 0, unroll=4)

            w0 = plsc.load_gather(wbuf, [(t * 2 + 0) * NS * L + sid * L + iot])
            w1 = plsc.load_gather(wbuf, [(t * 2 + 1) * NS * L + sid * L + iot])

            pltpu.sync_copy(ng.at[pl.ds(0, ACC_T)], acc)

            def ebody(j, cidx):
                work = []
                for bq in range(BQ):
                    ci = cidx + 2 * bq
                    spair = plsc.load_gather(win_s, [ci])
                    dpair = plsc.load_gather(win_d, [ci])
                    u = plsc.load_gather(u_tab, [dpair])
                    v = plsc.load_gather(v_tab, [dpair])
                    z = u * w0 + v * w1
                    sswap = _vperm(spair, swp)
                    eq = spair == sswap
                    zswap = _vperm(z, swp)
                    z2 = jnp.where(eq, jnp.maximum(z, zswap), z)
                    mask = jnp.logical_or(jnp.logical_not(eq), fh)
                    work.append((spair * 8 + coloff, z2, mask))
                for addr, z2, mask in work:
                    cur = plsc.load_gather(acc, [addr], mask=mask)
                    plsc.store_scatter(acc, [addr], jnp.maximum(cur, z2),
                                       mask=mask)
                return cidx + 2 * BQ

            def wbody(kw, _, s_hbm=s_hbm, d_hbm=d_hbm):
                b = cid * EH + kw * W
                pltpu.sync_copy(s_hbm.at[pl.ds(b, W)], win_s)
                pltpu.sync_copy(d_hbm.at[pl.ds(b, W)], win_d)
                lax.fori_loop(0, W // (2 * BQ), ebody, c0)
                return 0
            lax.fori_loop(0, nw, wbody, 0)

            blk = (cid * NS + sid) * BLK_T
            pltpu.sync_copy(acc, m_out.at[pl.ds(blk, ACC_T)])

    return k(p_ret, p_dram, p_lnk, wp1, src_ret, dst_ret, src_dram, dst_dram,
             src_lnk, dst_lnk, neg1)


def _sc_segmax_link(th1, src_lnk, dst_lnk, neg1):
    W = 8000
    nw = E // W

    @functools.partial(
        pl.kernel,
        out_type=jax.ShapeDtypeStruct((NC * NS * ACC_L,), jnp.float32),
        mesh=_MESH,
        scratch_types=[
            pltpu.VMEM((ACC_T,), jnp.float32),
            pltpu.VMEM((ACC_L,), jnp.float32),
            pltpu.VMEM((W,), jnp.int32),
            pltpu.VMEM((W,), jnp.int32),
        ],
        compiler_params=_SC_PARAMS,
    )
    def k(th, sl, dl, ng, mm, table, acc, win_s, win_d):
        cid = lax.axis_index("c")
        sid = lax.axis_index("s")
        iot = _iota()
        coloff = iot & 7
        c0 = iot >> 3
        fh = iot < 8
        swp = (iot + 8) & 15
        lo = jnp.full((L,), cid * NLH, jnp.int32)

        pltpu.sync_copy(th.at[pl.ds(sid * BLK_T, ACC_T)], table)
        pltpu.sync_copy(ng.at[pl.ds(0, ACC_L)], acc)

        def ebody(j, cidx):
            work = []
            for bq in range(BQ):
                ci = cidx + 2 * bq
                tpair = plsc.load_gather(win_s, [ci])
                lpair = plsc.load_gather(win_d, [ci])
                z = plsc.load_gather(table, [tpair * 8 + coloff])
                lswap = _vperm(lpair, swp)
                eq = lpair == lswap
                zswap = _vperm(z, swp)
                z2 = jnp.where(eq, jnp.maximum(z, zswap), z)
                row = lpair - lo
                inr = jnp.logical_and(row >= 0, row < NLH)
                rowc = jnp.clip(row, 0, NLH - 1)
                mask = jnp.logical_and(
                    inr, jnp.logical_or(jnp.logical_not(eq), fh))
                work.append((rowc * 8 + coloff, z2, mask))
            for addr, z2, mask in work:
                cur = plsc.load_gather(acc, [addr], mask=mask)
                plsc.store_scatter(acc, [addr], jnp.maximum(cur, z2),
                                   mask=mask)
            return cidx + 2 * BQ

        def wbody(kw, _):
            b = kw * W
            pltpu.sync_copy(sl.at[pl.ds(b, W)], win_s)
            pltpu.sync_copy(dl.at[pl.ds(b, W)], win_d)
            lax.fori_loop(0, W // (2 * BQ), ebody, c0)
            return 0
        lax.fori_loop(0, nw, wbody, 0)

        blk = (cid * NS + sid) * ACC_L
        pltpu.sync_copy(acc, mm.at[pl.ds(blk, ACC_L)])

    return k(th1, src_lnk, dst_lnk, neg1)


def _gelu_ln(y, g, beta):
    y = y * 0.5 * (1.0 + lax.erf(y * 0.7071067811865476))
    mu = jnp.mean(y, axis=-1, keepdims=True)
    var = jnp.mean((y - mu) ** 2, axis=-1, keepdims=True)
    return (y - mu) / jnp.sqrt(var + 1e-5) * g + beta


def _task_body(mr_ref, md_ref, ml_ref, br_ref, bd_ref, bl_ref,
               wt_ref, bt_ref, g_ref, beta_ref, o_ref):
    hs = []
    for m_ref, b_ref in ((mr_ref, br_ref), (md_ref, bd_ref), (ml_ref, bl_ref)):
        mm = jnp.maximum(m_ref[0], m_ref[1])
        hs.append(jnp.where(mm > _TCUT, jnp.tanh(mm + b_ref[...]), 0.0))
    t = jnp.concatenate(hs, axis=-1)
    y = t @ wt_ref[...] + bt_ref[...]
    o_ref[...] = _gelu_ln(y, g_ref[...], beta_ref[...])


def _tc_task(m_ret, m_dram, m_lnk, b_ret, b_dram, b_lnk,
             W_task, b_task, g_task, beta_task):
    R = 512
    vec = pl.BlockSpec((1, H), lambda i: (0, 0))
    return pl.pallas_call(
        _task_body,
        grid=(MT // R,),
        in_specs=[
            pl.BlockSpec((NC, R, H), lambda i: (0, i, 0)),
            pl.BlockSpec((NC, R, H), lambda i: (0, i, 0)),
            pl.BlockSpec((NC, R, H), lambda i: (0, i, 0)),
            vec, vec, vec,
            pl.BlockSpec((3 * H, H), lambda i: (0, 0)),
            vec, vec, vec,
        ],
        out_specs=pl.BlockSpec((R, H), lambda i: (i, 0)),
        out_shape=jax.ShapeDtypeStruct((MT, H), jnp.float32),
    )(m_ret, m_dram, m_lnk, b_ret, b_dram, b_lnk,
      W_task, b_task, g_task, beta_task)


def _link_body(m_ref, w_ref, b_ref, g_ref, beta_ref, o_ref):
    x = jnp.where(m_ref[...] > _TCUT, m_ref[...], 0.0)
    y = x @ w_ref[...] + b_ref[...]
    o_ref[...] = _gelu_ln(y, g_ref[...], beta_ref[...])


def _tc_link(m_mod, W_mod, b_mod, g_link, beta_link):
    R = 512
    vec = pl.BlockSpec((1, H), lambda i: (0, 0))
    return pl.pallas_call(
        _link_body,
        grid=(N_LINK // R,),
        in_specs=[
            pl.BlockSpec((R, H), lambda i: (i, 0)),
            pl.BlockSpec((H, H), lambda i: (0, 0)),
            vec, vec, vec,
        ],
        out_specs=pl.BlockSpec((R, H), lambda i: (i, 0)),
        out_shape=jax.ShapeDtypeStruct((N_LINK, H), jnp.float32),
    )(m_mod, W_mod, b_mod, g_link, beta_link)


def _wpair(w):
    return jnp.tile(w.reshape(2, NS, 8), (1, 1, 2))


def _asm_task(m1):
    m4 = m1.reshape(NC, NS, MT, 8)
    return m4.transpose(0, 2, 1, 3).reshape(NC, MT, H)


def kernel(feat_reticle, feat_dram_port, feat_link, src_reticle, dst_reticle,
           src_dram_port, dst_dram_port, src_link, dst_link,
           W_ret, b_ret, W_dram, b_dram, W_lnk, b_lnk,
           W_task, b_task, W_mod, b_mod,
           g_task, beta_task, g_link, beta_link):
    z1 = jnp.zeros((2048,), jnp.float32)
    neg1 = jnp.full((ACC_T,), _NEG, jnp.float32)
    wp1 = jnp.stack(
        [_wpair(W_ret), _wpair(W_dram), _wpair(W_lnk)]).reshape(-1)

    p_ret, p_dram, p_lnk = _sc_segsum(
        feat_reticle.reshape(-1), dst_reticle,
        feat_dram_port.reshape(-1), dst_dram_port,
        feat_link.reshape(-1), dst_link, z1)

    m1_ret, m1_dram, m1_lnk = _sc_segmax3(
        p_ret, p_dram, p_lnk, wp1, src_reticle, dst_reticle,
        src_dram_port, dst_dram_port, src_link, dst_link, neg1)

    task_h = _tc_task(
        _asm_task(m1_ret), _asm_task(m1_dram), _asm_task(m1_lnk),
        b_ret.reshape(1, H), b_dram.reshape(1, H), b_lnk.reshape(1, H),
        W_task, b_task.reshape(1, H), g_task.reshape(1, H),
        beta_task.reshape(1, H))

    th1 = task_h.reshape(MT, NS, 8).transpose(1, 0, 2).reshape(-1)

    mm1 = _sc_segmax_link(th1, src_link, dst_link, neg1)
    m_mod = mm1.reshape(NC, NS, NLH, 8).transpose(0, 2, 1, 3).reshape(
        N_LINK, H)

    link_h = _tc_link(
        m_mod, W_mod, b_mod.reshape(1, H), g_link.reshape(1, H),
        beta_link.reshape(1, H))
    return link_h
